# Initial kernel scaffold; baseline (speedup 1.0000x reference)
import jax
import jax.numpy as jnp
from jax.experimental import pallas as pl


def kernel(trajs, table):
    raise NotImplementedError("write your pallas kernel here")


# speedup vs baseline: 201.7167x; 201.7167x over previous
import functools

import jax
import jax.numpy as jnp
import numpy as np
from jax.experimental import pallas as pl

_B = 16384
_L = 200
_V = 21
_E = 128
_BR = 512


def _pe_mean() -> np.ndarray:
    pos = np.arange(_L, dtype=np.float32)
    ang = (2.0 * np.pi * pos / float(_L)).astype(np.float32)
    freqs = np.arange(1, _E // 2 + 1, dtype=np.float32)
    phase = ang[:, None] * freqs[None, :]
    pe = np.concatenate([np.sin(phase), np.cos(phase)], axis=-1)
    return pe.mean(axis=0).astype(np.float32)


_PE_MEAN = _pe_mean()


def _body(tr_ref, tab_ref, pe_ref, out_ref):
    t = tr_ref[...]
    cols = []
    for v in range(_V):
        m = (t == v).astype(jnp.float32)
        cols.append(jnp.sum(m, axis=1, keepdims=True))
    counts = jnp.concatenate(cols, axis=1)
    acc = jax.lax.dot_general(
        counts, tab_ref[...], (((1,), (0,)), ((), ())),
        preferred_element_type=jnp.float32)
    out_ref[...] = acc * (1.0 / _L) + pe_ref[0:1, :]


@jax.jit
def kernel(trajs, table):
    pe = jnp.broadcast_to(jnp.asarray(_PE_MEAN)[None, :], (8, _E))
    grid = (_B // _BR,)
    return pl.pallas_call(
        _body,
        grid=grid,
        in_specs=[
            pl.BlockSpec((_BR, _L), lambda i: (i, 0)),
            pl.BlockSpec((_V, _E), lambda i: (0, 0)),
            pl.BlockSpec((8, _E), lambda i: (0, 0)),
        ],
        out_specs=pl.BlockSpec((_BR, _E), lambda i: (i, 0)),
        out_shape=jax.ShapeDtypeStruct((_B, _E), jnp.float32),
    )(trajs.astype(jnp.int32), table, pe)
